# baseline (speedup 1.0000x reference)
import functools

import jax
import jax.numpy as jnp
from jax import lax
from jax.experimental import pallas as pl
from jax.experimental.pallas import tpu as pltpu
from jax.experimental.pallas import tpu_sc as plsc

_STATE = 128
_HID = 512
_ABS = 2048
_NAG = 32
_APA = 2

_TC_AG = 16
_SC_AG = _NAG - _TC_AG
_APS = 2
_CW = 1024
_KB = 16
_NK = _HID // _KB
_NEG = -3.0e38
_BIG = 1 << 30


def _fc1_kernel(state_ref, w1_ref, b1_ref, x_ref):
    xt = lax.dot_general(w1_ref[...], state_ref[...],
                         (((0,), (1,)), ((), ())),
                         preferred_element_type=jnp.float32)
    xt = jnp.maximum(xt + b1_ref[...], 0.0)
    x_ref[...] = jnp.broadcast_to(xt, (_HID, 128))


def _tc_shard_kernel(x_ref, w2_ref, b2_ref, g1_ref, g2_ref, m_ref, i_ref):
    x = x_ref[:, 0:1].reshape(1, _HID)
    y2 = jnp.dot(x, w2_ref[...], preferred_element_type=jnp.float32)
    for a in range(_APS):
        y = y2[:, a * _ABS:(a + 1) * _ABS]
        y = y + b2_ref[a] + g1_ref[a] + g2_ref[a]
        m = jnp.max(y, axis=1, keepdims=True)
        lane = lax.broadcasted_iota(jnp.int32, (1, _ABS), 1)
        idx = jnp.min(jnp.where(y == m, lane, _ABS), axis=1, keepdims=True)
        m_ref[a] = jnp.broadcast_to(m, (1, 128))
        i_ref[a] = jnp.broadcast_to(idx, (1, 128))


def _sc_shard_body(x_hbm, w2_hbm, b2_hbm, g1_hbm, g2_hbm, scm_hbm, sci_hbm,
                   xv, buf0, buf1, acc, gtmp, om, oi, sem0, sem1):
    nlanes = 16
    wid = lax.axis_index("s") * 2 + lax.axis_index("c")
    agent = _TC_AG + wid // 2
    half = wid % 2
    coff = half * _CW
    cbase = agent * _ABS + coff

    pltpu.sync_copy(x_hbm, xv)

    pltpu.sync_copy(b2_hbm.at[agent, pl.ds(coff, _CW)], acc)
    for g_hbm in (g1_hbm, g2_hbm):
        pltpu.sync_copy(g_hbm.at[agent, pl.ds(coff, _CW)], gtmp)

        def _addg(j, _):
            sl = pl.ds(j * nlanes, nlanes)
            acc[sl] = acc[sl] + gtmp[sl]
            return 0
        lax.fori_loop(0, _CW // nlanes, _addg, 0)

    bufs = (buf0, buf1)
    sems = (sem0, sem1)

    def _start(c):
        return pltpu.async_copy(
            w2_hbm.at[pl.ds(c * _KB, _KB), pl.ds(cbase, _CW)],
            bufs[c % 2], sems[c % 2])

    cur = _start(0)
    for c in range(_NK):
        nxt = _start(c + 1) if c + 1 < _NK else None
        cur.wait()
        cur = nxt
        buf = bufs[c % 2]
        xks = [xv[c * _KB + t, pl.ds(0, nlanes)] for t in range(_KB)]

        def _accum(j, _, buf=buf, xks=xks):
            sl = pl.ds(j * nlanes, nlanes)
            a = acc[sl]
            for t in range(_KB):
                a = a + xks[t] * buf[t, sl]
            acc[sl] = a
            return 0
        lax.fori_loop(0, _CW // nlanes, _accum, 0)

    m = jnp.full((nlanes,), _NEG, jnp.float32)
    bestj = jnp.zeros((nlanes,), jnp.int32)
    for j in range(_CW // nlanes):
        v = acc[pl.ds(j * nlanes, nlanes)]
        gt = v > m
        m = jnp.where(gt, v, m)
        bestj = jnp.where(gt, jnp.full((nlanes,), j, jnp.int32), bestj)
    lanei = lax.broadcasted_iota(jnp.int32, (nlanes,), 0)
    col = bestj * nlanes + lanei + coff
    om[pl.ds(0, nlanes)] = m
    oi[pl.ds(0, nlanes)] = col
    for j in range(1, 128 // nlanes):
        sl = pl.ds(j * nlanes, nlanes)
        om[sl] = jnp.full((nlanes,), _NEG, jnp.float32)
        oi[sl] = jnp.full((nlanes,), _BIG, jnp.int32)
    pltpu.sync_copy(om, scm_hbm.at[wid])
    pltpu.sync_copy(oi, sci_hbm.at[wid])


def _merge_kernel(tcm_ref, tci_ref, scm_ref, sci_ref, dm_ref, out_ref):
    tci = tci_ref[:, 0, :]
    panm, pani = [], []
    for p in range(2):
        sm = scm_ref[:, p, :]
        si = sci_ref[:, p, :]
        mp = jnp.max(sm, axis=1, keepdims=True)
        ip = jnp.min(jnp.where(sm == mp, si, _BIG), axis=1, keepdims=True)
        panm.append(mp)
        pani.append(ip)
    hi = panm[1] > panm[0]
    sci = jnp.broadcast_to(jnp.where(hi, pani[1], pani[0]), (_SC_AG, 128))
    idx = jnp.concatenate([tci, sci], axis=0)
    sel = lax.broadcasted_iota(jnp.int32, (_NAG, _ABS), 1) == idx[:, 0:1]
    for p in range(_APA):
        dmp = dm_ref[:, p, :]
        val = jnp.sum(jnp.where(sel, dmp, 0), axis=1, keepdims=True)
        out_ref[p] = jnp.broadcast_to(val, (_NAG, 128))


def kernel(state, W1, b1, W2, b2, g1, g2, decode_map):
    state2 = state.reshape(1, _STATE)
    b1r = b1.reshape(_HID, 1)
    b2r = b2.reshape(_NAG, _ABS)
    g1r = g1.reshape(_NAG, 1, _ABS)
    g2r = g2.reshape(_NAG, 1, _ABS)
    dm = decode_map.transpose(0, 2, 1)

    x = pl.pallas_call(
        _fc1_kernel,
        in_specs=[
            pl.BlockSpec((1, _STATE), lambda: (0, 0)),
            pl.BlockSpec((_STATE, _HID), lambda: (0, 0)),
            pl.BlockSpec((_HID, 1), lambda: (0, 0)),
        ],
        out_specs=pl.BlockSpec((_HID, 128), lambda: (0, 0)),
        out_shape=jax.ShapeDtypeStruct((_HID, 128), jnp.float32),
    )(state2, W1, b1r)

    tcm, tci = pl.pallas_call(
        _tc_shard_kernel,
        grid=(_TC_AG // _APS,),
        in_specs=[
            pl.BlockSpec((_HID, 128), lambda i: (0, 0)),
            pl.BlockSpec((_HID, _APS * _ABS), lambda i: (0, i)),
            pl.BlockSpec((_APS, 1, _ABS), lambda i: (i, 0, 0)),
            pl.BlockSpec((_APS, 1, _ABS), lambda i: (i, 0, 0)),
            pl.BlockSpec((_APS, 1, _ABS), lambda i: (i, 0, 0)),
        ],
        out_specs=[
            pl.BlockSpec((_APS, 1, 128), lambda i: (i, 0, 0)),
            pl.BlockSpec((_APS, 1, 128), lambda i: (i, 0, 0)),
        ],
        out_shape=[
            jax.ShapeDtypeStruct((_TC_AG, 1, 128), jnp.float32),
            jax.ShapeDtypeStruct((_TC_AG, 1, 128), jnp.int32),
        ],
        compiler_params=pltpu.CompilerParams(
            dimension_semantics=("arbitrary",)),
    )(x, W2, b2r.reshape(_NAG, 1, _ABS), g1r, g2r)

    sc_call = pl.kernel(
        _sc_shard_body,
        out_type=[
            jax.ShapeDtypeStruct((2 * _SC_AG, 128), jnp.float32),
            jax.ShapeDtypeStruct((2 * _SC_AG, 128), jnp.int32),
        ],
        mesh=plsc.VectorSubcoreMesh(core_axis_name="c", subcore_axis_name="s"),
        scratch_types=[
            pltpu.VMEM((_HID, 128), jnp.float32),
            pltpu.VMEM((_KB, _CW), jnp.float32),
            pltpu.VMEM((_KB, _CW), jnp.float32),
            pltpu.VMEM((_CW,), jnp.float32),
            pltpu.VMEM((_CW,), jnp.float32),
            pltpu.VMEM((128,), jnp.float32),
            pltpu.VMEM((128,), jnp.int32),
            pltpu.SemaphoreType.DMA,
            pltpu.SemaphoreType.DMA,
        ],
    )
    scm, sci = sc_call(x, W2, b2r, g1, g2)

    out = pl.pallas_call(
        _merge_kernel,
        in_specs=[
            pl.BlockSpec((_TC_AG, 1, 128), lambda: (0, 0, 0)),
            pl.BlockSpec((_TC_AG, 1, 128), lambda: (0, 0, 0)),
            pl.BlockSpec((_SC_AG, 2, 128), lambda: (0, 0, 0)),
            pl.BlockSpec((_SC_AG, 2, 128), lambda: (0, 0, 0)),
            pl.BlockSpec((_NAG, _APA, _ABS), lambda: (0, 0, 0)),
        ],
        out_specs=pl.BlockSpec((_APA, _NAG, 128), lambda: (0, 0, 0)),
        out_shape=jax.ShapeDtypeStruct((_APA, _NAG, 128), jnp.int32),
    )(tcm, tci, scm.reshape(_SC_AG, 2, 128), sci.reshape(_SC_AG, 2, 128), dm)

    return out[:, :, 0].T.reshape(-1)


# speedup vs baseline: 1.0921x; 1.0921x over previous
import functools

import jax
import jax.numpy as jnp
from jax import lax
from jax.experimental import pallas as pl
from jax.experimental.pallas import tpu as pltpu
from jax.experimental.pallas import tpu_sc as plsc

_STATE = 128
_HID = 512
_ABS = 2048
_NAG = 32
_APA = 2

_TC_AG = 16
_SC_AG = _NAG - _TC_AG
_APS = 2
_CW = 1024
_KB = 16
_NK = _HID // _KB
_NEG = -3.0e38
_BIG = 1 << 30


def _fc1_kernel(state_ref, w1_ref, b1_ref, x_ref):
    xt = lax.dot_general(w1_ref[...], state_ref[...],
                         (((0,), (1,)), ((), ())),
                         preferred_element_type=jnp.float32)
    xt = jnp.maximum(xt + b1_ref[...], 0.0)
    x_ref[...] = jnp.broadcast_to(xt, (_HID, 128))


def _tc_shard_kernel(x_ref, w2_ref, b2_ref, g1_ref, g2_ref, m_ref, i_ref):
    x = x_ref[:, 0:1].reshape(1, _HID)
    y2 = jnp.dot(x, w2_ref[...], preferred_element_type=jnp.float32)
    for a in range(_APS):
        y = y2[:, a * _ABS:(a + 1) * _ABS]
        y = y + b2_ref[a] + g1_ref[a] + g2_ref[a]
        m = jnp.max(y, axis=1, keepdims=True)
        lane = lax.broadcasted_iota(jnp.int32, (1, _ABS), 1)
        idx = jnp.min(jnp.where(y == m, lane, _ABS), axis=1, keepdims=True)
        m_ref[a] = jnp.broadcast_to(m, (1, 128))
        i_ref[a] = jnp.broadcast_to(idx, (1, 128))


def _sc_shard_body(x_hbm, w2_hbm, b2_hbm, g1_hbm, g2_hbm, scm_hbm, sci_hbm,
                   xv, buf0, buf1, acc, gtmp, om, oi, sem0, sem1):
    nlanes = 16
    wid = lax.axis_index("s") * 2 + lax.axis_index("c")
    agent = _TC_AG + wid // 2
    half = wid % 2
    coff = half * _CW
    cbase = agent * _ABS + coff

    pltpu.sync_copy(x_hbm, xv)

    pltpu.sync_copy(b2_hbm.at[agent, pl.ds(coff, _CW)], acc)
    for g_hbm in (g1_hbm, g2_hbm):
        pltpu.sync_copy(g_hbm.at[agent, pl.ds(coff, _CW)], gtmp)

        def _addg(j, _):
            sl = pl.ds(j * nlanes, nlanes)
            acc[sl] = acc[sl] + gtmp[sl]
            return 0
        lax.fori_loop(0, _CW // nlanes, _addg, 0)

    bufs = (buf0, buf1)
    sems = (sem0, sem1)

    def _start(c):
        return pltpu.async_copy(
            w2_hbm.at[pl.ds(c * _KB, _KB), pl.ds(cbase, _CW)],
            bufs[c % 2], sems[c % 2])

    cur = _start(0)
    for c in range(_NK):
        nxt = _start(c + 1) if c + 1 < _NK else None
        cur.wait()
        cur = nxt
        buf = bufs[c % 2]
        xks = [xv[c * _KB + t, pl.ds(0, nlanes)] for t in range(_KB)]

        def _accum(j, _, buf=buf, xks=xks):
            sl = pl.ds(j * nlanes, nlanes)
            p = [xks[t] * buf[t, sl] for t in range(4)]
            for t in range(4, _KB):
                p[t % 4] = p[t % 4] + xks[t] * buf[t, sl]
            acc[sl] = acc[sl] + ((p[0] + p[1]) + (p[2] + p[3]))
            return 0
        lax.fori_loop(0, _CW // nlanes, _accum, 0)

    m = jnp.full((nlanes,), _NEG, jnp.float32)
    bestj = jnp.zeros((nlanes,), jnp.int32)
    for j in range(_CW // nlanes):
        v = acc[pl.ds(j * nlanes, nlanes)]
        gt = v > m
        m = jnp.where(gt, v, m)
        bestj = jnp.where(gt, jnp.full((nlanes,), j, jnp.int32), bestj)
    lanei = lax.broadcasted_iota(jnp.int32, (nlanes,), 0)
    col = bestj * nlanes + lanei + coff
    om[pl.ds(0, nlanes)] = m
    oi[pl.ds(0, nlanes)] = col
    for j in range(1, 128 // nlanes):
        sl = pl.ds(j * nlanes, nlanes)
        om[sl] = jnp.full((nlanes,), _NEG, jnp.float32)
        oi[sl] = jnp.full((nlanes,), _BIG, jnp.int32)
    pltpu.sync_copy(om, scm_hbm.at[wid])
    pltpu.sync_copy(oi, sci_hbm.at[wid])


def _merge_kernel(tcm_ref, tci_ref, scm_ref, sci_ref, dm_ref, out_ref):
    tci = tci_ref[:, 0, :]
    panm, pani = [], []
    for p in range(2):
        sm = scm_ref[:, p, :]
        si = sci_ref[:, p, :]
        mp = jnp.max(sm, axis=1, keepdims=True)
        ip = jnp.min(jnp.where(sm == mp, si, _BIG), axis=1, keepdims=True)
        panm.append(mp)
        pani.append(ip)
    hi = panm[1] > panm[0]
    sci = jnp.broadcast_to(jnp.where(hi, pani[1], pani[0]), (_SC_AG, 128))
    idx = jnp.concatenate([tci, sci], axis=0)
    sel = lax.broadcasted_iota(jnp.int32, (_NAG, _ABS), 1) == idx[:, 0:1]
    for p in range(_APA):
        dmp = dm_ref[:, p, :]
        val = jnp.sum(jnp.where(sel, dmp, 0), axis=1, keepdims=True)
        out_ref[p] = jnp.broadcast_to(val, (_NAG, 128))


def kernel(state, W1, b1, W2, b2, g1, g2, decode_map):
    state2 = state.reshape(1, _STATE)
    b1r = b1.reshape(_HID, 1)
    b2r = b2.reshape(_NAG, _ABS)
    g1r = g1.reshape(_NAG, 1, _ABS)
    g2r = g2.reshape(_NAG, 1, _ABS)
    dm = decode_map.transpose(0, 2, 1)

    x = pl.pallas_call(
        _fc1_kernel,
        in_specs=[
            pl.BlockSpec((1, _STATE), lambda: (0, 0)),
            pl.BlockSpec((_STATE, _HID), lambda: (0, 0)),
            pl.BlockSpec((_HID, 1), lambda: (0, 0)),
        ],
        out_specs=pl.BlockSpec((_HID, 128), lambda: (0, 0)),
        out_shape=jax.ShapeDtypeStruct((_HID, 128), jnp.float32),
    )(state2, W1, b1r)

    sc_call = pl.kernel(
        _sc_shard_body,
        out_type=[
            jax.ShapeDtypeStruct((2 * _SC_AG, 128), jnp.float32),
            jax.ShapeDtypeStruct((2 * _SC_AG, 128), jnp.int32),
        ],
        mesh=plsc.VectorSubcoreMesh(core_axis_name="c", subcore_axis_name="s"),
        scratch_types=[
            pltpu.VMEM((_HID, 128), jnp.float32),
            pltpu.VMEM((_KB, _CW), jnp.float32),
            pltpu.VMEM((_KB, _CW), jnp.float32),
            pltpu.VMEM((_CW,), jnp.float32),
            pltpu.VMEM((_CW,), jnp.float32),
            pltpu.VMEM((128,), jnp.float32),
            pltpu.VMEM((128,), jnp.int32),
            pltpu.SemaphoreType.DMA,
            pltpu.SemaphoreType.DMA,
        ],
    )
    scm, sci = sc_call(x, W2, b2r, g1, g2)

    tcm, tci = pl.pallas_call(
        _tc_shard_kernel,
        grid=(_TC_AG // _APS,),
        in_specs=[
            pl.BlockSpec((_HID, 128), lambda i: (0, 0)),
            pl.BlockSpec((_HID, _APS * _ABS), lambda i: (0, i)),
            pl.BlockSpec((_APS, 1, _ABS), lambda i: (i, 0, 0)),
            pl.BlockSpec((_APS, 1, _ABS), lambda i: (i, 0, 0)),
            pl.BlockSpec((_APS, 1, _ABS), lambda i: (i, 0, 0)),
        ],
        out_specs=[
            pl.BlockSpec((_APS, 1, 128), lambda i: (i, 0, 0)),
            pl.BlockSpec((_APS, 1, 128), lambda i: (i, 0, 0)),
        ],
        out_shape=[
            jax.ShapeDtypeStruct((_TC_AG, 1, 128), jnp.float32),
            jax.ShapeDtypeStruct((_TC_AG, 1, 128), jnp.int32),
        ],
        compiler_params=pltpu.CompilerParams(
            dimension_semantics=("arbitrary",)),
    )(x, W2, b2r.reshape(_NAG, 1, _ABS), g1r, g2r)


    out = pl.pallas_call(
        _merge_kernel,
        in_specs=[
            pl.BlockSpec((_TC_AG, 1, 128), lambda: (0, 0, 0)),
            pl.BlockSpec((_TC_AG, 1, 128), lambda: (0, 0, 0)),
            pl.BlockSpec((_SC_AG, 2, 128), lambda: (0, 0, 0)),
            pl.BlockSpec((_SC_AG, 2, 128), lambda: (0, 0, 0)),
            pl.BlockSpec((_NAG, _APA, _ABS), lambda: (0, 0, 0)),
        ],
        out_specs=pl.BlockSpec((_APA, _NAG, 128), lambda: (0, 0, 0)),
        out_shape=jax.ShapeDtypeStruct((_APA, _NAG, 128), jnp.int32),
    )(tcm, tci, scm.reshape(_SC_AG, 2, 128), sci.reshape(_SC_AG, 2, 128), dm)

    return out[:, :, 0].T.reshape(-1)


# speedup vs baseline: 1.2179x; 1.1152x over previous
import functools

import jax
import jax.numpy as jnp
from jax import lax
from jax.experimental import pallas as pl
from jax.experimental.pallas import tpu as pltpu
from jax.experimental.pallas import tpu_sc as plsc

_STATE = 128
_HID = 512
_ABS = 2048
_NAG = 32
_APA = 2

_TC_AG = 16
_SC_AG = _NAG - _TC_AG
_APS = 2
_CW = 1024
_KB = 32
_NK = _HID // _KB
_NEG = -3.0e38
_BIG = 1 << 30


def _fc1_kernel(state_ref, w1_ref, b1_ref, m_ref, r_ref, x_ref):
    xt = lax.dot_general(w1_ref[...], state_ref[...],
                         (((0,), (1,)), ((), ())),
                         preferred_element_type=jnp.float32)
    xt = jnp.maximum(xt + b1_ref[...], 0.0)
    g = xt * m_ref[...]
    x_ref[...] = jnp.dot(r_ref[...], g, preferred_element_type=jnp.float32)


def _tc_shard_kernel(state_ref, w1_ref, b1_ref, w2_ref, b2_ref, g1_ref,
                     g2_ref, m_ref, i_ref):
    x = jnp.maximum(
        jnp.dot(state_ref[...], w1_ref[...],
                preferred_element_type=jnp.float32) + b1_ref[...], 0.0)
    y2 = jnp.dot(x, w2_ref[...], preferred_element_type=jnp.float32)
    for a in range(_APS):
        y = y2[:, a * _ABS:(a + 1) * _ABS]
        y = y + b2_ref[a] + g1_ref[a] + g2_ref[a]
        m = jnp.max(y, axis=1, keepdims=True)
        lane = lax.broadcasted_iota(jnp.int32, (1, _ABS), 1)
        idx = jnp.min(jnp.where(y == m, lane, _ABS), axis=1, keepdims=True)
        m_ref[a] = jnp.broadcast_to(m, (1, 128))
        i_ref[a] = jnp.broadcast_to(idx, (1, 128))


def _sc_shard_body(x_hbm, w2_hbm, b2_hbm, g1_hbm, g2_hbm, scm_hbm, sci_hbm,
                   xv, buf0, buf1, buf2, acc, gtmp, om, oi,
                   sem0, sem1, sem2):
    nlanes = 16
    wid = lax.axis_index("s") * 2 + lax.axis_index("c")
    agent = _TC_AG + wid // 2
    half = wid % 2
    coff = half * _CW
    cbase = agent * _ABS + coff

    pltpu.sync_copy(x_hbm, xv)

    pltpu.sync_copy(b2_hbm.at[agent, pl.ds(coff, _CW)], acc)
    for g_hbm in (g1_hbm, g2_hbm):
        pltpu.sync_copy(g_hbm.at[agent, pl.ds(coff, _CW)], gtmp)

        def _addg(j, _):
            sl = pl.ds(j * nlanes, nlanes)
            acc[sl] = acc[sl] + gtmp[sl]
            return 0
        lax.fori_loop(0, _CW // nlanes, _addg, 0)

    bufs = (buf0, buf1, buf2)
    sems = (sem0, sem1, sem2)

    def _start(c):
        return pltpu.async_copy(
            w2_hbm.at[pl.ds(c * _KB, _KB), pl.ds(cbase, _CW)],
            bufs[c % 3], sems[c % 3])

    descs = {c: _start(c) for c in range(min(2, _NK))}
    for c in range(_NK):
        if c + 2 < _NK:
            descs[c + 2] = _start(c + 2)
        descs.pop(c).wait()
        buf = bufs[c % 3]
        xks = [xv[(c * _KB + t) // 8, pl.ds(((c * _KB + t) % 8) * nlanes,
                                            nlanes)] for t in range(_KB)]

        def _accum(j, _, buf=buf, xks=xks):
            sl = pl.ds(j * nlanes, nlanes)
            p = [xks[t] * buf[t, sl] for t in range(4)]
            for t in range(4, _KB):
                p[t % 4] = p[t % 4] + xks[t] * buf[t, sl]
            acc[sl] = acc[sl] + ((p[0] + p[1]) + (p[2] + p[3]))
            return 0
        lax.fori_loop(0, _CW // nlanes, _accum, 0)

    m = jnp.full((nlanes,), _NEG, jnp.float32)
    bestj = jnp.zeros((nlanes,), jnp.int32)
    for j in range(_CW // nlanes):
        v = acc[pl.ds(j * nlanes, nlanes)]
        gt = v > m
        m = jnp.where(gt, v, m)
        bestj = jnp.where(gt, jnp.full((nlanes,), j, jnp.int32), bestj)
    lanei = lax.broadcasted_iota(jnp.int32, (nlanes,), 0)
    col = bestj * nlanes + lanei + coff
    om[pl.ds(0, nlanes)] = m
    oi[pl.ds(0, nlanes)] = col
    for j in range(1, 128 // nlanes):
        sl = pl.ds(j * nlanes, nlanes)
        om[sl] = jnp.full((nlanes,), _NEG, jnp.float32)
        oi[sl] = jnp.full((nlanes,), _BIG, jnp.int32)
    pltpu.sync_copy(om, scm_hbm.at[wid])
    pltpu.sync_copy(oi, sci_hbm.at[wid])


def _merge_kernel(tcm_ref, tci_ref, scm_ref, sci_ref, dm_ref, out_ref):
    tci = tci_ref[:, 0, :]
    panm, pani = [], []
    for p in range(2):
        sm = scm_ref[:, p, :]
        si = sci_ref[:, p, :]
        mp = jnp.max(sm, axis=1, keepdims=True)
        ip = jnp.min(jnp.where(sm == mp, si, _BIG), axis=1, keepdims=True)
        panm.append(mp)
        pani.append(ip)
    hi = panm[1] > panm[0]
    sci = jnp.broadcast_to(jnp.where(hi, pani[1], pani[0]), (_SC_AG, 128))
    idx = jnp.concatenate([tci, sci], axis=0)
    sel = lax.broadcasted_iota(jnp.int32, (_NAG, _ABS), 1) == idx[:, 0:1]
    for p in range(_APA):
        dmp = dm_ref[:, p, :]
        val = jnp.sum(jnp.where(sel, dmp, 0), axis=1, keepdims=True)
        out_ref[p] = jnp.broadcast_to(val, (_NAG, 128))


def kernel(state, W1, b1, W2, b2, g1, g2, decode_map):
    state2 = state.reshape(1, _STATE)
    b1r = b1.reshape(_HID, 1)
    b2r = b2.reshape(_NAG, _ABS)
    g1r = g1.reshape(_NAG, 1, _ABS)
    g2r = g2.reshape(_NAG, 1, _ABS)
    dm = decode_map.transpose(0, 2, 1)

    kk = jnp.arange(_HID)
    msel = ((jnp.arange(128)[None, :] // 16) == (kk[:, None] % 8)
            ).astype(jnp.float32)
    rsel = ((kk[None, :] // 8) == jnp.arange(_HID // 8)[:, None]
            ).astype(jnp.float32)
    x = pl.pallas_call(
        _fc1_kernel,
        in_specs=[
            pl.BlockSpec((1, _STATE), lambda: (0, 0)),
            pl.BlockSpec((_STATE, _HID), lambda: (0, 0)),
            pl.BlockSpec((_HID, 1), lambda: (0, 0)),
            pl.BlockSpec((_HID, 128), lambda: (0, 0)),
            pl.BlockSpec((_HID // 8, _HID), lambda: (0, 0)),
        ],
        out_specs=pl.BlockSpec((_HID // 8, 128), lambda: (0, 0)),
        out_shape=jax.ShapeDtypeStruct((_HID // 8, 128), jnp.float32),
    )(state2, W1, b1r, msel, rsel)

    sc_call = pl.kernel(
        _sc_shard_body,
        out_type=[
            jax.ShapeDtypeStruct((2 * _SC_AG, 128), jnp.float32),
            jax.ShapeDtypeStruct((2 * _SC_AG, 128), jnp.int32),
        ],
        mesh=plsc.VectorSubcoreMesh(core_axis_name="c", subcore_axis_name="s"),
        scratch_types=[
            pltpu.VMEM((_HID // 8, 128), jnp.float32),
            pltpu.VMEM((_KB, _CW), jnp.float32),
            pltpu.VMEM((_KB, _CW), jnp.float32),
            pltpu.VMEM((_KB, _CW), jnp.float32),
            pltpu.VMEM((_CW,), jnp.float32),
            pltpu.VMEM((_CW,), jnp.float32),
            pltpu.VMEM((128,), jnp.float32),
            pltpu.VMEM((128,), jnp.int32),
            pltpu.SemaphoreType.DMA,
            pltpu.SemaphoreType.DMA,
            pltpu.SemaphoreType.DMA,
        ],
    )
    scm, sci = sc_call(x, W2, b2r, g1, g2)

    tcm, tci = pl.pallas_call(
        _tc_shard_kernel,
        grid=(_TC_AG // _APS,),
        in_specs=[
            pl.BlockSpec((1, _STATE), lambda i: (0, 0)),
            pl.BlockSpec((_STATE, _HID), lambda i: (0, 0)),
            pl.BlockSpec((1, _HID), lambda i: (0, 0)),
            pl.BlockSpec((_HID, _APS * _ABS), lambda i: (0, i)),
            pl.BlockSpec((_APS, 1, _ABS), lambda i: (i, 0, 0)),
            pl.BlockSpec((_APS, 1, _ABS), lambda i: (i, 0, 0)),
            pl.BlockSpec((_APS, 1, _ABS), lambda i: (i, 0, 0)),
        ],
        out_specs=[
            pl.BlockSpec((_APS, 1, 128), lambda i: (i, 0, 0)),
            pl.BlockSpec((_APS, 1, 128), lambda i: (i, 0, 0)),
        ],
        out_shape=[
            jax.ShapeDtypeStruct((_TC_AG, 1, 128), jnp.float32),
            jax.ShapeDtypeStruct((_TC_AG, 1, 128), jnp.int32),
        ],
        compiler_params=pltpu.CompilerParams(
            dimension_semantics=("arbitrary",)),
    )(state2, W1, b1.reshape(1, _HID), W2, b2r.reshape(_NAG, 1, _ABS),
      g1r, g2r)


    out = pl.pallas_call(
        _merge_kernel,
        in_specs=[
            pl.BlockSpec((_TC_AG, 1, 128), lambda: (0, 0, 0)),
            pl.BlockSpec((_TC_AG, 1, 128), lambda: (0, 0, 0)),
            pl.BlockSpec((_SC_AG, 2, 128), lambda: (0, 0, 0)),
            pl.BlockSpec((_SC_AG, 2, 128), lambda: (0, 0, 0)),
            pl.BlockSpec((_NAG, _APA, _ABS), lambda: (0, 0, 0)),
        ],
        out_specs=pl.BlockSpec((_APA, _NAG, 128), lambda: (0, 0, 0)),
        out_shape=jax.ShapeDtypeStruct((_APA, _NAG, 128), jnp.int32),
    )(tcm, tci, scm.reshape(_SC_AG, 2, 128), sci.reshape(_SC_AG, 2, 128), dm)

    return out[:, :, 0].T.reshape(-1)


# speedup vs baseline: 1.3272x; 1.0898x over previous
import jax
import jax.numpy as jnp
from jax import lax
from jax.experimental import pallas as pl
from jax.experimental.pallas import tpu as pltpu
from jax.experimental.pallas import tpu_sc as plsc

_STATE = 128
_HID = 512
_ABS = 2048
_NAG = 32
_APA = 2

_SC_AG = 8
_TC_AG = _NAG - _SC_AG
_TPA = 32 // _SC_AG
_CW = _ABS // _TPA
_APS = 2
_KB = 32
_NK = _HID // _KB
_NBUF = 4
_NEG = -3.0e38
_BIG = 1 << 30
_LAN = 16


def _fc1_kernel(state_ref, w1_ref, b1_ref, m_ref, r_ref, x_ref):
    xt = lax.dot_general(w1_ref[...], state_ref[...],
                         (((0,), (1,)), ((), ())),
                         preferred_element_type=jnp.float32)
    xt = jnp.maximum(xt + b1_ref[...], 0.0)
    g = xt * m_ref[...]
    x_ref[...] = jnp.dot(r_ref[...], g, preferred_element_type=jnp.float32)


def _tc_shard_kernel(state_ref, w1_ref, b1_ref, w2_ref, b2_ref, g1_ref,
                     g2_ref, m_ref, i_ref):
    x = jnp.maximum(
        jnp.dot(state_ref[...], w1_ref[...],
                preferred_element_type=jnp.float32) + b1_ref[...], 0.0)
    y2 = jnp.dot(x, w2_ref[...], preferred_element_type=jnp.float32)
    for a in range(_APS):
        y = y2[:, a * _ABS:(a + 1) * _ABS]
        y = y + b2_ref[a] + g1_ref[a] + g2_ref[a]
        m = jnp.max(y, axis=1, keepdims=True)
        lane = lax.broadcasted_iota(jnp.int32, (1, _ABS), 1)
        idx = jnp.min(jnp.where(y == m, lane, _ABS), axis=1, keepdims=True)
        m_ref[a] = jnp.broadcast_to(m, (1, 128))
        i_ref[a] = jnp.broadcast_to(idx, (1, 128))


def _sc_shard_body(x_hbm, w2_hbm, b2_hbm, g1_hbm, g2_hbm, scm_hbm, sci_hbm,
                   xv, bufs, acc, gtmp, om, oi, sems):
    wid = lax.axis_index("s") * 2 + lax.axis_index("c")
    agent = _TC_AG + wid // _TPA
    part = wid % _TPA
    coff = part * _CW
    cbase = agent * _ABS + coff

    pltpu.sync_copy(x_hbm, xv)

    pltpu.sync_copy(b2_hbm.at[agent, pl.ds(coff, _CW)], acc)
    for g_hbm in (g1_hbm, g2_hbm):
        pltpu.sync_copy(g_hbm.at[agent, pl.ds(coff, _CW)], gtmp)

        def _addg(j, _):
            sl = pl.ds(j * _LAN, _LAN)
            acc[sl] = acc[sl] + gtmp[sl]
            return 0
        lax.fori_loop(0, _CW // _LAN, _addg, 0)

    def _start(c):
        return pltpu.async_copy(
            w2_hbm.at[pl.ds(c * _KB, _KB), pl.ds(cbase, _CW)],
            bufs[c % _NBUF], sems[c % _NBUF])

    descs = {c: _start(c) for c in range(min(_NBUF - 1, _NK))}
    for c in range(_NK):
        if c + _NBUF - 1 < _NK:
            descs[c + _NBUF - 1] = _start(c + _NBUF - 1)
        descs.pop(c).wait()
        buf = bufs[c % _NBUF]
        xks = [xv[(c * _KB + t) // 8, pl.ds(((c * _KB + t) % 8) * _LAN,
                                            _LAN)] for t in range(_KB)]

        def _accum(j, _, buf=buf, xks=xks):
            sl = pl.ds(j * _LAN, _LAN)
            p = [xks[t] * buf[t, sl] for t in range(4)]
            for t in range(4, _KB):
                p[t % 4] = p[t % 4] + xks[t] * buf[t, sl]
            acc[sl] = acc[sl] + ((p[0] + p[1]) + (p[2] + p[3]))
            return 0
        lax.fori_loop(0, _CW // _LAN, _accum, 0)

    m = jnp.full((_LAN,), _NEG, jnp.float32)
    bestj = jnp.zeros((_LAN,), jnp.int32)
    for j in range(_CW // _LAN):
        v = acc[pl.ds(j * _LAN, _LAN)]
        gt = v > m
        m = jnp.where(gt, v, m)
        bestj = jnp.where(gt, jnp.full((_LAN,), j, jnp.int32), bestj)
    lanei = lax.broadcasted_iota(jnp.int32, (_LAN,), 0)
    col = bestj * _LAN + lanei + coff
    om[pl.ds(0, _LAN)] = m
    oi[pl.ds(0, _LAN)] = col
    for j in range(1, 128 // _LAN):
        sl = pl.ds(j * _LAN, _LAN)
        om[sl] = jnp.full((_LAN,), _NEG, jnp.float32)
        oi[sl] = jnp.full((_LAN,), _BIG, jnp.int32)
    pltpu.sync_copy(om, scm_hbm.at[wid])
    pltpu.sync_copy(oi, sci_hbm.at[wid])


def _sc_entry(x_hbm, w2_hbm, b2_hbm, g1_hbm, g2_hbm, scm_hbm, sci_hbm,
              xv, b0, b1_, b2_, b3_, acc, gtmp, om, oi, s0, s1, s2, s3):
    _sc_shard_body(x_hbm, w2_hbm, b2_hbm, g1_hbm, g2_hbm, scm_hbm, sci_hbm,
                   xv, (b0, b1_, b2_, b3_), acc, gtmp, om, oi,
                   (s0, s1, s2, s3))


def _merge_kernel(tcm_ref, tci_ref, scm_ref, sci_ref, dm_ref, out_ref):
    tci = tci_ref[:, 0, :]
    bm = bi = None
    for p in range(_TPA):
        sm = scm_ref[:, p, :]
        si = sci_ref[:, p, :]
        mp = jnp.max(sm, axis=1, keepdims=True)
        ip = jnp.min(jnp.where(sm == mp, si, _BIG), axis=1, keepdims=True)
        if bm is None:
            bm, bi = mp, ip
        else:
            hi = mp > bm
            bm = jnp.where(hi, mp, bm)
            bi = jnp.where(hi, ip, bi)
    sci = jnp.broadcast_to(bi, (_SC_AG, 128))
    idx = jnp.concatenate([tci, sci], axis=0)
    sel = lax.broadcasted_iota(jnp.int32, (_NAG, _ABS), 1) == idx[:, 0:1]
    for p in range(_APA):
        dmp = dm_ref[:, p, :]
        val = jnp.sum(jnp.where(sel, dmp, 0), axis=1, keepdims=True)
        out_ref[p] = jnp.broadcast_to(val, (_NAG, 128))


def kernel(state, W1, b1, W2, b2, g1, g2, decode_map):
    state2 = state.reshape(1, _STATE)
    b1r = b1.reshape(_HID, 1)
    b2r = b2.reshape(_NAG, _ABS)
    g1r = g1.reshape(_NAG, 1, _ABS)
    g2r = g2.reshape(_NAG, 1, _ABS)
    dm = decode_map.transpose(0, 2, 1)

    kk = jnp.arange(_HID)
    msel = ((jnp.arange(128)[None, :] // 16) == (kk[:, None] % 8)
            ).astype(jnp.float32)
    rsel = ((kk[None, :] // 8) == jnp.arange(_HID // 8)[:, None]
            ).astype(jnp.float32)
    x = pl.pallas_call(
        _fc1_kernel,
        in_specs=[
            pl.BlockSpec((1, _STATE), lambda: (0, 0)),
            pl.BlockSpec((_STATE, _HID), lambda: (0, 0)),
            pl.BlockSpec((_HID, 1), lambda: (0, 0)),
            pl.BlockSpec((_HID, 128), lambda: (0, 0)),
            pl.BlockSpec((_HID // 8, _HID), lambda: (0, 0)),
        ],
        out_specs=pl.BlockSpec((_HID // 8, 128), lambda: (0, 0)),
        out_shape=jax.ShapeDtypeStruct((_HID // 8, 128), jnp.float32),
    )(state2, W1, b1r, msel, rsel)

    sc_call = pl.kernel(
        _sc_entry,
        out_type=[
            jax.ShapeDtypeStruct((32, 128), jnp.float32),
            jax.ShapeDtypeStruct((32, 128), jnp.int32),
        ],
        mesh=plsc.VectorSubcoreMesh(core_axis_name="c", subcore_axis_name="s"),
        scratch_types=(
            [pltpu.VMEM((_HID // 8, 128), jnp.float32)]
            + [pltpu.VMEM((_KB, _CW), jnp.float32) for _ in range(_NBUF)]
            + [pltpu.VMEM((_CW,), jnp.float32),
               pltpu.VMEM((_CW,), jnp.float32),
               pltpu.VMEM((128,), jnp.float32),
               pltpu.VMEM((128,), jnp.int32)]
            + [pltpu.SemaphoreType.DMA for _ in range(_NBUF)]
        ),
    )
    scm, sci = sc_call(x, W2, b2r, g1, g2)

    tcm, tci = pl.pallas_call(
        _tc_shard_kernel,
        grid=(_TC_AG // _APS,),
        in_specs=[
            pl.BlockSpec((1, _STATE), lambda i: (0, 0)),
            pl.BlockSpec((_STATE, _HID), lambda i: (0, 0)),
            pl.BlockSpec((1, _HID), lambda i: (0, 0)),
            pl.BlockSpec((_HID, _APS * _ABS), lambda i: (0, i)),
            pl.BlockSpec((_APS, 1, _ABS), lambda i: (i, 0, 0)),
            pl.BlockSpec((_APS, 1, _ABS), lambda i: (i, 0, 0)),
            pl.BlockSpec((_APS, 1, _ABS), lambda i: (i, 0, 0)),
        ],
        out_specs=[
            pl.BlockSpec((_APS, 1, 128), lambda i: (i, 0, 0)),
            pl.BlockSpec((_APS, 1, 128), lambda i: (i, 0, 0)),
        ],
        out_shape=[
            jax.ShapeDtypeStruct((_TC_AG, 1, 128), jnp.float32),
            jax.ShapeDtypeStruct((_TC_AG, 1, 128), jnp.int32),
        ],
        compiler_params=pltpu.CompilerParams(
            dimension_semantics=("arbitrary",)),
    )(state2, W1, b1.reshape(1, _HID), W2, b2r.reshape(_NAG, 1, _ABS),
      g1r, g2r)

    out = pl.pallas_call(
        _merge_kernel,
        in_specs=[
            pl.BlockSpec((_TC_AG, 1, 128), lambda: (0, 0, 0)),
            pl.BlockSpec((_TC_AG, 1, 128), lambda: (0, 0, 0)),
            pl.BlockSpec((_SC_AG, _TPA, 128), lambda: (0, 0, 0)),
            pl.BlockSpec((_SC_AG, _TPA, 128), lambda: (0, 0, 0)),
            pl.BlockSpec((_NAG, _APA, _ABS), lambda: (0, 0, 0)),
        ],
        out_specs=pl.BlockSpec((_APA, _NAG, 128), lambda: (0, 0, 0)),
        out_shape=jax.ShapeDtypeStruct((_APA, _NAG, 128), jnp.int32),
    )(tcm, tci, scm.reshape(_SC_AG, _TPA, 128),
      sci.reshape(_SC_AG, _TPA, 128), dm)

    return out[:, :, 0].T.reshape(-1)


# speedup vs baseline: 2.0461x; 1.5417x over previous
import jax
import jax.numpy as jnp
from jax import lax
from jax.experimental import pallas as pl
from jax.experimental.pallas import tpu as pltpu

_STATE = 128
_HID = 512
_ABS = 2048
_NAG = 32
_APA = 2

_APS = 2
_HAG = _NAG // 2
_GRID = _HAG // _APS


def _fused_kernel(state_ref, w1_ref, b1_ref, w2a_ref, w2b_ref,
                  b2a_ref, b2b_ref, g1a_ref, g1b_ref, g2a_ref, g2b_ref,
                  dma_ref, dmb_ref, outa_ref, outb_ref):
    x = jnp.maximum(
        jnp.dot(state_ref[...], w1_ref[...],
                preferred_element_type=jnp.float32) + b1_ref[...], 0.0)
    for w2_ref, b2_ref, g1_ref, g2_ref, dm_ref, out_ref in (
            (w2a_ref, b2a_ref, g1a_ref, g2a_ref, dma_ref, outa_ref),
            (w2b_ref, b2b_ref, g1b_ref, g2b_ref, dmb_ref, outb_ref)):
        y2 = jnp.dot(x, w2_ref[...], preferred_element_type=jnp.float32)
        for a in range(_APS):
            y = y2[:, a * _ABS:(a + 1) * _ABS]
            y = y + b2_ref[a] + g1_ref[a] + g2_ref[a]
            m = jnp.max(y, axis=1, keepdims=True)
            lane = lax.broadcasted_iota(jnp.int32, (1, _ABS), 1)
            idx = jnp.min(jnp.where(y == m, lane, _ABS), axis=1,
                          keepdims=True)
            dm = dm_ref[a]
            lane2 = lax.broadcasted_iota(jnp.int32, (_APA, _ABS), 1)
            vals = jnp.sum(jnp.where(lane2 == idx, dm, 0), axis=1,
                           keepdims=True)
            out_ref[a] = jnp.broadcast_to(vals, (_APA, 128))


def kernel(state, W1, b1, W2, b2, g1, g2, decode_map):
    state2 = state.reshape(1, _STATE)
    b12 = b1.reshape(1, _HID)
    b2r = b2.reshape(_NAG, 1, _ABS)
    g1r = g1.reshape(_NAG, 1, _ABS)
    g2r = g2.reshape(_NAG, 1, _ABS)
    dm = decode_map.transpose(0, 2, 1)

    small = [
        pl.BlockSpec((_APS, 1, _ABS), lambda i: (i, 0, 0)),
        pl.BlockSpec((_APS, 1, _ABS), lambda i: (i + _GRID, 0, 0)),
    ]
    outa, outb = pl.pallas_call(
        _fused_kernel,
        grid=(_GRID,),
        in_specs=[
            pl.BlockSpec((1, _STATE), lambda i: (0, 0)),
            pl.BlockSpec((_STATE, _HID), lambda i: (0, 0)),
            pl.BlockSpec((1, _HID), lambda i: (0, 0)),
            pl.BlockSpec((_HID, _APS * _ABS), lambda i: (0, i)),
            pl.BlockSpec((_HID, _APS * _ABS), lambda i: (0, i + _GRID)),
            small[0], small[1],
            small[0], small[1],
            small[0], small[1],
            pl.BlockSpec((_APS, _APA, _ABS), lambda i: (i, 0, 0)),
            pl.BlockSpec((_APS, _APA, _ABS), lambda i: (i + _GRID, 0, 0)),
        ],
        out_specs=[
            pl.BlockSpec((_APS, _APA, 128), lambda i: (i, 0, 0)),
            pl.BlockSpec((_APS, _APA, 128), lambda i: (i, 0, 0)),
        ],
        out_shape=[
            jax.ShapeDtypeStruct((_HAG, _APA, 128), jnp.int32),
            jax.ShapeDtypeStruct((_HAG, _APA, 128), jnp.int32),
        ],
        compiler_params=pltpu.CompilerParams(
            dimension_semantics=("arbitrary",)),
    )(state2, W1, b12, W2, W2, b2r, b2r, g1r, g1r, g2r, g2r, dm, dm)
    out = jnp.concatenate([outa, outb], axis=0)
    return out[:, :, 0].reshape(-1)
